# Initial kernel scaffold; baseline (speedup 1.0000x reference)
import jax
import jax.numpy as jnp
from jax.experimental import pallas as pl


def kernel(nxyz, bonds, bond_len, num_bonds):
    raise NotImplementedError("write your pallas kernel here")


# speedup vs baseline: 13.2756x; 13.2756x over previous
import functools

import jax
import jax.numpy as jnp
from jax import lax
from jax.experimental import pallas as pl
from jax.experimental.pallas import tpu as pltpu
from jax.experimental.pallas import tpu_sc as plsc

N = 50000
E = 319600
K_SPRING = 20.0

NC = 2
NS = 16
NW = NC * NS
PB = 128
NSTEP = 80
CHUNK = NSTEP * PB
E_PAD = NW * CHUNK
ROWS_PER_TILE = 3128
N_PAD = NS * ROWS_PER_TILE


def _bond_body(xcol, ycol, zcol, src_hbm, dst_hbm, r0_hbm, zeros_hbm,
               e_out, gxo, gyo, gzo,
               src_v, dst_v, r0_v,
               xs_b, ys_b, zs_b, xd_b, yd_b, zd_b,
               gxp, gyp, gzp, gxn, gyn, gzn,
               eacc, zbuf, gx_sh, gy_sh, gz_sh):
    cid = lax.axis_index("c")
    sid = lax.axis_index("s")
    wid = cid * NS + sid

    pltpu.sync_copy(src_hbm.at[wid], src_v)
    pltpu.sync_copy(dst_hbm.at[wid], dst_v)
    pltpu.sync_copy(r0_hbm.at[pl.ds(wid * CHUNK, CHUNK)], r0_v)

    sl = pl.ds(sid * ROWS_PER_TILE, ROWS_PER_TILE)
    pltpu.sync_copy(zeros_hbm, zbuf)
    pltpu.sync_copy(zbuf, gx_sh.at[sl])
    pltpu.sync_copy(zbuf, gy_sh.at[sl])
    pltpu.sync_copy(zbuf, gz_sh.at[sl])

    eacc[...] = jnp.zeros((16,), jnp.float32)
    plsc.subcore_barrier()

    lane = lax.iota(jnp.int32, 16)
    base_bond = wid * CHUNK
    kf = jnp.float32(K_SPRING)
    k2 = jnp.float32(2.0 * K_SPRING)

    def step(j, carry):
        idx_s = src_v.at[j]
        idx_d = dst_v.at[j]
        pltpu.sync_copy(xcol.at[idx_s], xs_b)
        pltpu.sync_copy(ycol.at[idx_s], ys_b)
        pltpu.sync_copy(zcol.at[idx_s], zs_b)
        pltpu.sync_copy(xcol.at[idx_d], xd_b)
        pltpu.sync_copy(ycol.at[idx_d], yd_b)
        pltpu.sync_copy(zcol.at[idx_d], zd_b)
        for g in range(PB // 16):
            s16 = pl.ds(g * 16, 16)
            dx = xs_b[s16] - xd_b[s16]
            dy = ys_b[s16] - yd_b[s16]
            dz = zs_b[s16] - zd_b[s16]
            r2 = dx * dx + dy * dy + dz * dz
            i = lax.bitcast_convert_type(r2, jnp.int32)
            i = jnp.int32(0x5F3759DF) - lax.shift_right_logical(i, 1)
            y = lax.bitcast_convert_type(i, jnp.float32)
            h = jnp.float32(0.5) * r2
            for _ in range(3):
                y = y * (jnp.float32(1.5) - h * y * y)
            r = r2 * y
            diff = r - r0_v[pl.ds(j * PB + g * 16, 16)]
            valid = (base_bond + j * PB + g * 16 + lane) < E
            e = jnp.where(valid, kf * diff * diff, jnp.float32(0.0))
            eacc[...] = eacc[...] + e
            c = jnp.where(valid, k2 * diff * y, jnp.float32(0.0))
            gx = c * dx
            gy = c * dy
            gz = c * dz
            gxp[s16] = gx
            gyp[s16] = gy
            gzp[s16] = gz
            gxn[s16] = -gx
            gyn[s16] = -gy
            gzn[s16] = -gz
        pltpu.sync_copy(gxp, gx_sh.at[idx_s], add=True)
        pltpu.sync_copy(gyp, gy_sh.at[idx_s], add=True)
        pltpu.sync_copy(gzp, gz_sh.at[idx_s], add=True)
        pltpu.sync_copy(gxn, gx_sh.at[idx_d], add=True)
        pltpu.sync_copy(gyn, gy_sh.at[idx_d], add=True)
        pltpu.sync_copy(gzn, gz_sh.at[idx_d], add=True)
        return carry

    lax.fori_loop(0, NSTEP, step, 0)

    plsc.subcore_barrier()
    osl = pl.ds(cid * N_PAD + sid * ROWS_PER_TILE, ROWS_PER_TILE)
    pltpu.sync_copy(gx_sh.at[sl], zbuf)
    pltpu.sync_copy(zbuf, gxo.at[osl])
    pltpu.sync_copy(gy_sh.at[sl], zbuf)
    pltpu.sync_copy(zbuf, gyo.at[osl])
    pltpu.sync_copy(gz_sh.at[sl], zbuf)
    pltpu.sync_copy(zbuf, gzo.at[osl])
    pltpu.sync_copy(eacc, e_out.at[pl.ds(wid * 16, 16)])


@functools.partial(
    pl.kernel,
    mesh=plsc.VectorSubcoreMesh(core_axis_name="c", subcore_axis_name="s"),
    out_type=[
        jax.ShapeDtypeStruct((NW * 16,), jnp.float32),
        jax.ShapeDtypeStruct((NC * N_PAD,), jnp.float32),
        jax.ShapeDtypeStruct((NC * N_PAD,), jnp.float32),
        jax.ShapeDtypeStruct((NC * N_PAD,), jnp.float32),
    ],
    scratch_types=[
        pltpu.VMEM((NSTEP, PB), jnp.int32),
        pltpu.VMEM((NSTEP, PB), jnp.int32),
        pltpu.VMEM((CHUNK,), jnp.float32),
        pltpu.VMEM((PB,), jnp.float32),
        pltpu.VMEM((PB,), jnp.float32),
        pltpu.VMEM((PB,), jnp.float32),
        pltpu.VMEM((PB,), jnp.float32),
        pltpu.VMEM((PB,), jnp.float32),
        pltpu.VMEM((PB,), jnp.float32),
        pltpu.VMEM((PB,), jnp.float32),
        pltpu.VMEM((PB,), jnp.float32),
        pltpu.VMEM((PB,), jnp.float32),
        pltpu.VMEM((PB,), jnp.float32),
        pltpu.VMEM((PB,), jnp.float32),
        pltpu.VMEM((PB,), jnp.float32),
        pltpu.VMEM((16,), jnp.float32),
        pltpu.VMEM((ROWS_PER_TILE,), jnp.float32),
        pltpu.VMEM_SHARED((N_PAD,), jnp.float32),
        pltpu.VMEM_SHARED((N_PAD,), jnp.float32),
        pltpu.VMEM_SHARED((N_PAD,), jnp.float32),
    ],
)
def _bond_sc(xcol, ycol, zcol, src, dst, r0, zeros, e_out, gxo, gyo, gzo,
             *scratch):
    _bond_body(xcol, ycol, zcol, src, dst, r0, zeros, e_out, gxo, gyo, gzo,
               *scratch)


def kernel(nxyz, bonds, bond_len, num_bonds):
    del num_bonds
    pad = E_PAD - E
    src = jnp.concatenate([bonds[:, 0], jnp.zeros((pad,), jnp.int32)])
    dst = jnp.concatenate([bonds[:, 1], jnp.ones((pad,), jnp.int32)])
    r0 = jnp.concatenate([bond_len[:, 0], jnp.ones((pad,), jnp.float32)])
    src = src.reshape(NW, NSTEP, PB)
    dst = dst.reshape(NW, NSTEP, PB)
    xcol = nxyz[:, 1]
    ycol = nxyz[:, 2]
    zcol = nxyz[:, 3]
    zeros = jnp.zeros((ROWS_PER_TILE,), jnp.float32)
    e_parts, gxo, gyo, gzo = _bond_sc(xcol, ycol, zcol, src, dst, r0, zeros)
    energy = jnp.sum(e_parts)
    gx = gxo[:N] + gxo[N_PAD:N_PAD + N]
    gy = gyo[:N] + gyo[N_PAD:N_PAD + N]
    gz = gzo[:N] + gzo[N_PAD:N_PAD + N]
    grad = jnp.stack([gx, gy, gz], axis=1)
    return energy, grad
